# baseline (speedup 1.0000x reference)
import functools

import jax
import jax.numpy as jnp
from jax import lax
from jax.experimental import pallas as pl
from jax.experimental.pallas import tpu as pltpu
from jax.experimental.pallas import tpu_sc as plsc

VOCAB = 1000000
EMBED_DIM = 32
PACKED = EMBED_DIM // 2
BATCH = 16384

NC = 2
NS = 16
L = 16
NW = NC * NS
B_PER_W = BATCH // NW
CHUNK = 128
N_CHUNKS = B_PER_W // CHUNK
GROUPS = CHUNK // L


def _sc_body(emb_hbm, tgt_hbm, ctx_hbm, out_hbm,
             idx_tv, idx_cv, rows_t, rows_c, out_v, sem_t, sem_c):
    wid = lax.axis_index("s") * NC + lax.axis_index("c")
    base = wid * B_PER_W

    pltpu.sync_copy(tgt_hbm.at[pl.ds(base, B_PER_W)], idx_tv)
    pltpu.sync_copy(ctx_hbm.at[pl.ds(base, B_PER_W)], idx_cv)

    def fire(j, buf):
        def issue(g, _):
            vt = idx_tv[pl.ds(j * CHUNK + g * L, L)]
            vc = idx_cv[pl.ds(j * CHUNK + g * L, L)]
            for r in range(L):
                p = g * L + r
                pltpu.async_copy(
                    emb_hbm.at[pl.ds(vt[r], 1)],
                    rows_t.at[buf, pl.ds(p, 1)], sem_t)
                pltpu.async_copy(
                    emb_hbm.at[pl.ds(vc[r], 1)],
                    rows_c.at[buf, pl.ds(p, 1)], sem_c)
            return 0
        lax.fori_loop(0, GROUPS, issue, 0)

    def drain(buf):
        pltpu.make_async_copy(
            emb_hbm.at[pl.ds(0, CHUNK)], rows_t.at[buf], sem_t).wait()
        pltpu.make_async_copy(
            emb_hbm.at[pl.ds(0, CHUNK)], rows_c.at[buf], sem_c).wait()

    lanes = jnp.arange(L, dtype=jnp.int32)
    fire(0, 0)
    for j in range(N_CHUNKS):
        if j + 1 < N_CHUNKS:
            fire(j + 1, (j + 1) % 2)
        buf = j % 2
        drain(buf)
        for g in range(GROUPS):
            pos = g * L + lanes
            acc = jnp.zeros((L,), jnp.float32)
            for h in range(PACKED):
                colv = jnp.full((L,), h, jnp.int32)
                wt = plsc.load_gather(rows_t.at[buf], [pos, colv])
                wc = plsc.load_gather(rows_c.at[buf], [pos, colv])
                t_lo, t_hi = plsc.unpack(plsc.bitcast(wt, jnp.bfloat16),
                                         format=plsc.PackFormat.INTERLEAVED)
                c_lo, c_hi = plsc.unpack(plsc.bitcast(wc, jnp.bfloat16),
                                         format=plsc.PackFormat.INTERLEAVED)
                acc = acc + t_lo * c_lo + t_hi * c_hi
            out_v[pl.ds(j * CHUNK + g * L, L)] = acc

    pltpu.sync_copy(out_v, out_hbm.at[pl.ds(base, B_PER_W)])


@jax.jit
def _word2vec_score(target_word, context_word, embeddings):
    emb_packed = jax.lax.bitcast_convert_type(
        embeddings.astype(jnp.bfloat16).reshape(VOCAB, PACKED, 2),
        jnp.int32)
    mesh = plsc.VectorSubcoreMesh(core_axis_name="c", subcore_axis_name="s")
    k = functools.partial(
        pl.kernel,
        mesh=mesh,
        compiler_params=pltpu.CompilerParams(needs_layout_passes=False),
        out_type=jax.ShapeDtypeStruct((BATCH,), jnp.float32),
        scratch_types=[
            pltpu.VMEM((B_PER_W,), jnp.int32),
            pltpu.VMEM((B_PER_W,), jnp.int32),
            pltpu.VMEM((2, CHUNK, PACKED), jnp.int32),
            pltpu.VMEM((2, CHUNK, PACKED), jnp.int32),
            pltpu.VMEM((B_PER_W,), jnp.float32),
            pltpu.SemaphoreType.DMA,
            pltpu.SemaphoreType.DMA,
        ],
    )(_sc_body)
    return k(emb_packed, target_word, context_word)


def kernel(target_word, context_word, embeddings):
    return _word2vec_score(target_word.astype(jnp.int32),
                           context_word.astype(jnp.int32),
                           embeddings)


# speedup vs baseline: 2.7695x; 2.7695x over previous
import functools

import jax
import jax.numpy as jnp
from jax import lax
from jax.experimental import pallas as pl
from jax.experimental.pallas import tpu as pltpu
from jax.experimental.pallas import tpu_sc as plsc

VOCAB = 1000000
EMBED_DIM = 32
BATCH = 16384

NC = 2
NS = 16
L = 16
NW = NC * NS
B_PER_W = BATCH // NW
CHUNK = 128
N_CHUNKS = B_PER_W // CHUNK
GROUPS = CHUNK // L


def _sc_body(emb_hbm, tgt_hbm, ctx_hbm, out_hbm,
             idx_tv, idx_cv, rows_t, rows_c, out_v, sem_t, sem_c):
    wid = lax.axis_index("s") * NC + lax.axis_index("c")
    base = wid * B_PER_W

    pltpu.sync_copy(tgt_hbm.at[pl.ds(base, B_PER_W)], idx_tv)
    pltpu.sync_copy(ctx_hbm.at[pl.ds(base, B_PER_W)], idx_cv)

    def fire(j, buf):
        def issue(g, _):
            vt = idx_tv[pl.ds(j * CHUNK + g * L, L)]
            vc = idx_cv[pl.ds(j * CHUNK + g * L, L)]
            for r in range(L):
                pltpu.async_copy(emb_hbm.at[pl.ds(vt[r], 1)],
                                 rows_t.at[buf, pl.ds(g * L + r, 1)], sem_t)
                pltpu.async_copy(emb_hbm.at[pl.ds(vc[r], 1)],
                                 rows_c.at[buf, pl.ds(g * L + r, 1)], sem_c)
            return 0
        lax.fori_loop(0, GROUPS, issue, 0)

    def drain(buf):
        pltpu.make_async_copy(
            emb_hbm.at[pl.ds(0, CHUNK)], rows_t.at[buf], sem_t).wait()
        pltpu.make_async_copy(
            emb_hbm.at[pl.ds(0, CHUNK)], rows_c.at[buf], sem_c).wait()

    lanes = jnp.arange(L, dtype=jnp.int32)
    fire(0, 0)
    for j in range(N_CHUNKS):
        if j + 1 < N_CHUNKS:
            fire(j + 1, (j + 1) % 2)
        buf = j % 2
        drain(buf)
        for g in range(GROUPS):
            pos = g * L + lanes
            acc = jnp.zeros((L,), jnp.float32)
            for col in range(EMBED_DIM):
                colv = jnp.full((L,), col, jnp.int32)
                vt = plsc.load_gather(rows_t.at[buf], [pos, colv])
                vc = plsc.load_gather(rows_c.at[buf], [pos, colv])
                acc = acc + vt * vc
            out_v[pl.ds(j * CHUNK + g * L, L)] = acc

    pltpu.sync_copy(out_v, out_hbm.at[pl.ds(base, B_PER_W)])


@jax.jit
def _word2vec_score(target_word, context_word, embeddings):
    mesh = plsc.VectorSubcoreMesh(core_axis_name="c", subcore_axis_name="s")
    k = functools.partial(
        pl.kernel,
        mesh=mesh,
        compiler_params=pltpu.CompilerParams(needs_layout_passes=False),
        out_type=jax.ShapeDtypeStruct((BATCH,), jnp.float32),
        scratch_types=[
            pltpu.VMEM((B_PER_W,), jnp.int32),
            pltpu.VMEM((B_PER_W,), jnp.int32),
            pltpu.VMEM((2, CHUNK, EMBED_DIM), jnp.float32),
            pltpu.VMEM((2, CHUNK, EMBED_DIM), jnp.float32),
            pltpu.VMEM((B_PER_W,), jnp.float32),
            pltpu.SemaphoreType.DMA,
            pltpu.SemaphoreType.DMA,
        ],
    )(_sc_body)
    return k(embeddings, target_word, context_word)


def kernel(target_word, context_word, embeddings):
    return _word2vec_score(target_word.astype(jnp.int32),
                           context_word.astype(jnp.int32),
                           embeddings)


# speedup vs baseline: 3.8991x; 1.4079x over previous
import functools

import jax
import jax.numpy as jnp
from jax import lax
from jax.experimental import pallas as pl
from jax.experimental.pallas import tpu as pltpu
from jax.experimental.pallas import tpu_sc as plsc

VOCAB = 1000000
EMBED_DIM = 32
BATCH = 16384

NC = 2
NS = 16
L = 16
NW = NC * NS
B_PER_W = BATCH // NW
GROUPS = B_PER_W // L
WAVE = 4
N_WAVES = L // WAVE


def _sc_body(emb_hbm, tgt_hbm, ctx_hbm, out_hbm,
             idx_tv, idx_cv, blk_t, blk_c, out_v, sem_t, sem_c):
    wid = lax.axis_index("s") * NC + lax.axis_index("c")
    base = wid * B_PER_W

    pltpu.sync_copy(tgt_hbm.at[pl.ds(base, B_PER_W)], idx_tv)
    pltpu.sync_copy(ctx_hbm.at[pl.ds(base, B_PER_W)], idx_cv)

    lanes = jnp.arange(L, dtype=jnp.int32)
    comps_lo = jnp.arange(L, dtype=jnp.int32)
    comps_hi = comps_lo + L

    def group_body(g, carry):
        vt = idx_tv[pl.ds(g * L, L)]
        vc = idx_cv[pl.ds(g * L, L)]

        def fire(w):
            buf = w % 2
            for k in range(WAVE):
                r = w * WAVE + k
                jt = pl.multiple_of((vt[r] >> 7) * 128, 128)
                jc = pl.multiple_of((vc[r] >> 7) * 128, 128)
                pltpu.async_copy(emb_hbm.at[:, pl.ds(jt, 128)],
                                 blk_t.at[buf, k], sem_t)
                pltpu.async_copy(emb_hbm.at[:, pl.ds(jc, 128)],
                                 blk_c.at[buf, k], sem_c)

        def drain():
            pltpu.make_async_copy(emb_hbm.at[:, pl.ds(0, WAVE * 128)],
                                  blk_t.at[0], sem_t).wait()
            pltpu.make_async_copy(emb_hbm.at[:, pl.ds(0, WAVE * 128)],
                                  blk_c.at[0], sem_c).wait()

        def compute(w, acc):
            buf = w % 2
            for k in range(WAVE):
                r = w * WAVE + k
                lt = jnp.full((L,), vt[r] & 127, jnp.int32)
                lc = jnp.full((L,), vc[r] & 127, jnp.int32)
                ta = plsc.load_gather(blk_t.at[buf, k], [comps_lo, lt])
                tb = plsc.load_gather(blk_t.at[buf, k], [comps_hi, lt])
                ca = plsc.load_gather(blk_c.at[buf, k], [comps_lo, lc])
                cb = plsc.load_gather(blk_c.at[buf, k], [comps_hi, lc])
                acc = jnp.where(lanes == r, jnp.sum(ta * ca + tb * cb), acc)
            return acc

        acc = jnp.zeros((L,), jnp.float32)
        fire(0)
        for w in range(N_WAVES):
            if w + 1 < N_WAVES:
                fire(w + 1)
            drain()
            acc = compute(w, acc)

        out_v[pl.ds(g * L, L)] = acc
        return carry

    lax.fori_loop(0, GROUPS, group_body, 0)

    pltpu.sync_copy(out_v, out_hbm.at[pl.ds(base, B_PER_W)])


@jax.jit
def _word2vec_score(target_word, context_word, embeddings):
    emb_t = embeddings.T
    mesh = plsc.VectorSubcoreMesh(core_axis_name="c", subcore_axis_name="s")
    k = functools.partial(
        pl.kernel,
        mesh=mesh,
        compiler_params=pltpu.CompilerParams(needs_layout_passes=False),
        out_type=jax.ShapeDtypeStruct((BATCH,), jnp.float32),
        scratch_types=[
            pltpu.VMEM((B_PER_W,), jnp.int32),
            pltpu.VMEM((B_PER_W,), jnp.int32),
            pltpu.VMEM((2, WAVE, EMBED_DIM, 128), jnp.float32),
            pltpu.VMEM((2, WAVE, EMBED_DIM, 128), jnp.float32),
            pltpu.VMEM((B_PER_W,), jnp.float32),
            pltpu.SemaphoreType.DMA,
            pltpu.SemaphoreType.DMA,
        ],
    )(_sc_body)
    return k(emb_t, target_word, context_word)


def kernel(target_word, context_word, embeddings):
    return _word2vec_score(target_word.astype(jnp.int32),
                           context_word.astype(jnp.int32),
                           embeddings)
